# baseline (speedup 1.0000x reference)
import functools
import jax
import jax.numpy as jnp
from jax import lax
from jax.experimental import pallas as pl
from jax.experimental.pallas import tpu as pltpu
from jax.experimental.pallas import tpu_sc as plsc

NC = 2
NS = 16
L = 16
CHUNK = 128
K_HOPS = 3

_MESH = dict(core_axis_name="c", subcore_axis_name="s",
             num_cores=NC, num_subcores=NS)
_CP = dict(needs_layout_passes=False, use_tc_tiling_on_sc=False)


def _full16(v):
    return jnp.full((L,), v, dtype=jnp.int32)


def _rsqrt_pos(x):
    bits = plsc.bitcast(x, jnp.int32)
    y = plsc.bitcast(jnp.int32(0x5F3759DF) - (bits >> 1), jnp.float32)
    for _ in range(3):
        y = y * (1.5 - 0.5 * x * y * y)
    return jnp.where(x > 0, y, 0.0)


def _eas_body(ea_ref, out_ref):
    sel = (lax.broadcasted_iota(jnp.int32, (128, 32), 0) // 4
           == lax.broadcasted_iota(jnp.int32, (128, 32), 1)).astype(jnp.float32)
    out_ref[...] = jnp.dot(ea_ref[...], sel, preferred_element_type=jnp.float32)


def kernel(x, edge_index, edge_attr):
    n, d = x.shape
    e = edge_index.shape[1]
    dh = d // NC
    row = edge_index[0]
    col = edge_index[1]

    ept = ((e + NS * CHUNK - 1) // (NS * CHUNK)) * CHUNK
    nchunks = ept // CHUNK
    nchunks = ((nchunks + 15) // 16) * 16
    ept = nchunks * CHUNK
    e_pad = ept * NS
    nch_sc = nchunks // 2
    npt = ((n + NS * L - 1) // (NS * L)) * L
    n_pad = npt * NS

    row_p = jnp.pad(row, (0, e_pad - e)).reshape(NS, nchunks, CHUNK)
    col_p = jnp.pad(col, (0, e_pad - e)).reshape(NS, nchunks, CHUNK)
    ea_p = jnp.pad(edge_attr, ((0, e_pad - e), (0, 0)))
    rem = n % CHUNK
    nfull = n - rem

    eas = pl.pallas_call(
        _eas_body,
        out_shape=jax.ShapeDtypeStruct((e_pad // 32, 32), jnp.float32),
    )(ea_p.reshape(e_pad // 32, 128))
    eas_w = eas.reshape(NS, nchunks, CHUNK)

    def _prep_body(row_hbm, col_hbm, eas_hbm, w_hbm,
                   col_v, eas_v, row_v, dis_v, w_v, sml, acc_sh, dis_sh):
        cid = lax.axis_index("c")
        sid = lax.axis_index("s")

        def zf(i, _):
            sml[pl.ds(i * L, L)] = jnp.zeros((L,), jnp.float32)
            return ()
        lax.fori_loop(0, npt // L, zf, (), unroll=False)
        pltpu.sync_copy(sml, acc_sh.at[pl.ds(sid * npt, npt)])
        plsc.subcore_barrier()

        pltpu.sync_copy(col_hbm.at[sid], col_v)
        pltpu.sync_copy(eas_hbm.at[sid], eas_v)

        def dbody(j, _):
            pltpu.sync_copy(eas_v.at[j], acc_sh.at[col_v.at[j]], add=True)
            return ()
        lax.fori_loop(0, nchunks, dbody, (), unroll=False)
        plsc.subcore_barrier()

        pltpu.sync_copy(acc_sh.at[pl.ds(sid * npt, npt)], sml)

        def rbody(i, _):
            sl = pl.ds(i * L, L)
            sml[sl] = _rsqrt_pos(sml[sl])
            return ()
        lax.fori_loop(0, npt // L, rbody, (), unroll=False)
        pltpu.sync_copy(sml, dis_sh.at[pl.ds(sid * npt, npt)])
        plsc.subcore_barrier()
        pltpu.sync_copy(dis_sh, dis_v)

        cbase = cid * nch_sc
        pltpu.sync_copy(row_hbm.at[sid].at[pl.ds(cbase, nch_sc)], row_v)

        def wbody(j, _):
            for g in range(CHUNK // L):
                sl = pl.ds(g * L, L)
                r16 = row_v[j, sl]
                c16 = col_v[cbase + j, sl]
                dr = plsc.load_gather(dis_v, [r16])
                dc = plsc.load_gather(dis_v, [c16])
                w_v[j, sl] = dr * eas_v[cbase + j, sl] * dc
            return ()
        lax.fori_loop(0, nch_sc, wbody, (), unroll=False)
        pltpu.sync_copy(w_v, w_hbm.at[sid].at[pl.ds(cbase, nch_sc)])

    w = pl.kernel(
        _prep_body,
        out_type=jax.ShapeDtypeStruct((NS, nchunks, CHUNK), jnp.float32),
        mesh=plsc.VectorSubcoreMesh(**_MESH),
        compiler_params=pltpu.CompilerParams(**_CP),
        scratch_types=[
            pltpu.VMEM((nchunks, CHUNK), jnp.int32),
            pltpu.VMEM((nchunks, CHUNK), jnp.float32),
            pltpu.VMEM((nch_sc, CHUNK), jnp.int32),
            pltpu.VMEM((n_pad,), jnp.float32),
            pltpu.VMEM((nch_sc, CHUNK), jnp.float32),
            pltpu.VMEM((npt,), jnp.float32),
            pltpu.VMEM_SHARED((n_pad,), jnp.float32),
            pltpu.VMEM_SHARED((n_pad,), jnp.float32),
        ],
    )(row_p, col_p, eas_w)

    NBUF = 4
    NSEG = 2
    nch_seg = nchunks // NSEG

    def _hops_body(x_hbm, row_hbm, col_hbm, w_hbm, split_hbm, final_hbm,
                   row_v, col_v, w_v, gbufs, sbufs, gsems, ssems, acc_sh):
        cid = lax.axis_index("c")
        sid = lax.axis_index("s")
        cs = pl.ds(cid * dh, dh)
        ie = lax.broadcasted_iota(jnp.int32, (L,), 0) * 2
        io = ie + 1

        def to_bf16(sbuf, gbuf):
            @plsc.parallel_loop(0, CHUNK, step=1, unroll=8)
            def _(r):
                for dd in range(dh // (2 * L)):
                    a = plsc.load_gather(sbuf, [_full16(r), ie + dd * 2 * L])
                    b = plsc.load_gather(sbuf, [_full16(r), io + dd * 2 * L])
                    gbuf[r, pl.ds(dd * 2 * L, 2 * L)] = plsc.pack(
                        a, b, format=plsc.PackFormat.INTERLEAVED)

        for i in range(npt // CHUNK):
            start = sid * npt + i * CHUNK

            @pl.when(start + CHUNK <= n)
            def _():
                pltpu.sync_copy(x_hbm.at[pl.ds(start, CHUNK), cs], sbufs[0])
                to_bf16(sbufs[0], gbufs[0])
                pltpu.sync_copy(gbufs[0],
                                split_hbm.at[0, cid, pl.ds(start, CHUNK)])
                pltpu.sync_copy(sbufs[0],
                                final_hbm.at[0, pl.ds(start, CHUNK), cs])
            if rem:
                @pl.when(start == nfull)
                def _():
                    pltpu.sync_copy(x_hbm.at[pl.ds(start, rem), cs],
                                    sbufs[0].at[pl.ds(0, rem)])
                    to_bf16(sbufs[0], gbufs[0])
                    pltpu.sync_copy(gbufs[0].at[pl.ds(0, rem)],
                                    split_hbm.at[0, cid, pl.ds(start, rem)])
                    pltpu.sync_copy(sbufs[0].at[pl.ds(0, rem)],
                                    final_hbm.at[0, pl.ds(start, rem), cs])
        plsc.subcore_barrier()

        def scale(j, gbuf, sbuf):
            @plsc.parallel_loop(0, CHUNK, step=1, unroll=16)
            def _(r):
                wb = plsc.load_gather(w_v, [_full16(j), _full16(r)])
                for dd in range(dh // (2 * L)):
                    v = gbuf[r, pl.ds(dd * 2 * L, 2 * L)]
                    a, b = plsc.unpack(v, format=plsc.PackFormat.INTERLEAVED)
                    plsc.store_scatter(
                        sbuf, [_full16(r), ie + dd * 2 * L], a * wb)
                    plsc.store_scatter(
                        sbuf, [_full16(r), io + dd * 2 * L], b * wb)

        def hop(k, _):
            src = split_hbm.at[k - 1].at[cid]

            def zfill(i, _):
                for g in range(dh // L):
                    sbufs[0][i, pl.ds(g * L, L)] = jnp.zeros((L,),
                                                             jnp.float32)
                return ()
            lax.fori_loop(0, CHUNK, zfill, (), unroll=False)

            def zbody(i, _):
                pltpu.sync_copy(
                    sbufs[0], acc_sh.at[pl.ds(sid * npt + i * CHUNK, CHUNK)])
                return ()
            lax.fori_loop(0, npt // CHUNK, zbody, (), unroll=False)
            plsc.subcore_barrier()

            for seg in range(NSEG):
                cbase = seg * nch_seg
                pltpu.sync_copy(row_hbm.at[sid].at[pl.ds(cbase, nch_seg)],
                                row_v)
                pltpu.sync_copy(col_hbm.at[sid].at[pl.ds(cbase, nch_seg)],
                                col_v)
                pltpu.sync_copy(w_hbm.at[sid].at[pl.ds(cbase, nch_seg)], w_v)

                for b in range(NBUF - 1):
                    pltpu.async_copy(src.at[row_v.at[b]], gbufs[b], gsems[b])

                def quad(jj, _):
                    for b in range(NBUF):
                        j = jj * NBUF + b
                        pb = (b + NBUF - 1) % NBUF

                        if b == 0:
                            pltpu.async_copy(
                                src.at[row_v.at[j + NBUF - 1]], gbufs[pb],
                                gsems[pb])
                        else:
                            @pl.when(jj < nch_seg // NBUF - 1)
                            def _():
                                pltpu.async_copy(
                                    src.at[row_v.at[j + NBUF - 1]],
                                    gbufs[pb], gsems[pb])

                        pltpu.make_async_copy(
                            src.at[row_v.at[j]], gbufs[b], gsems[b]).wait()

                        @pl.when(jj >= 1)
                        def _():
                            pltpu.make_async_copy(
                                sbufs[b], acc_sh.at[col_v.at[j - NBUF]],
                                ssems[b]).wait()

                        scale(j, gbufs[b], sbufs[b])
                        pltpu.async_copy(sbufs[b], acc_sh.at[col_v.at[j]],
                                         ssems[b], add=True)
                    return ()
                lax.fori_loop(0, nch_seg // NBUF, quad, (), unroll=False)
                for b in range(NBUF):
                    pltpu.make_async_copy(
                        sbufs[b],
                        acc_sh.at[col_v.at[nch_seg - NBUF + b]],
                        ssems[b]).wait()

            plsc.subcore_barrier()

            for i in range(npt // CHUNK):
                start = sid * npt + i * CHUNK
                pltpu.sync_copy(acc_sh.at[pl.ds(start, CHUNK)], sbufs[0])

                @pl.when(k < K_HOPS)
                def _():
                    to_bf16(sbufs[0], gbufs[0])
                    pltpu.sync_copy(gbufs[0],
                                    split_hbm.at[k, cid, pl.ds(start, CHUNK)])

                @pl.when(start + CHUNK <= n)
                def _():
                    pltpu.sync_copy(sbufs[0],
                                    final_hbm.at[k, pl.ds(start, CHUNK), cs])
                if rem:
                    @pl.when(start == nfull)
                    def _():
                        pltpu.sync_copy(
                            sbufs[0].at[pl.ds(0, rem)],
                            final_hbm.at[k, pl.ds(start, rem), cs])
            plsc.subcore_barrier()
            return ()
        lax.fori_loop(1, K_HOPS + 1, hop, (), unroll=False)

    _split, final = pl.kernel(
        _hops_body,
        out_type=(
            jax.ShapeDtypeStruct((K_HOPS, NC, n_pad, dh), jnp.bfloat16),
            jax.ShapeDtypeStruct((K_HOPS + 1, n, d), jnp.float32),
        ),
        mesh=plsc.VectorSubcoreMesh(**_MESH),
        compiler_params=pltpu.CompilerParams(**_CP),
        scratch_types=[
            pltpu.VMEM((nch_seg, CHUNK), jnp.int32),
            pltpu.VMEM((nch_seg, CHUNK), jnp.int32),
            pltpu.VMEM((nch_seg, CHUNK), jnp.float32),
            [pltpu.VMEM((CHUNK, dh), jnp.bfloat16) for _ in range(NBUF)],
            [pltpu.VMEM((CHUNK, dh), jnp.float32) for _ in range(NBUF)],
            [pltpu.SemaphoreType.DMA for _ in range(NBUF)],
            [pltpu.SemaphoreType.DMA for _ in range(NBUF)],
            pltpu.VMEM_SHARED((n_pad, dh), jnp.float32),
        ],
    )(x, row_p, col_p, w)
    return final


# speedup vs baseline: 1.0911x; 1.0911x over previous
import functools
import jax
import jax.numpy as jnp
from jax import lax
from jax.experimental import pallas as pl
from jax.experimental.pallas import tpu as pltpu
from jax.experimental.pallas import tpu_sc as plsc

NC = 2
NS = 16
L = 16
CHUNK = 128
K_HOPS = 3

_MESH = dict(core_axis_name="c", subcore_axis_name="s",
             num_cores=NC, num_subcores=NS)
_CP = dict(needs_layout_passes=False, use_tc_tiling_on_sc=False)


def _full16(v):
    return jnp.full((L,), v, dtype=jnp.int32)


def _rsqrt_pos(x):
    bits = plsc.bitcast(x, jnp.int32)
    y = plsc.bitcast(jnp.int32(0x5F3759DF) - (bits >> 1), jnp.float32)
    for _ in range(3):
        y = y * (1.5 - 0.5 * x * y * y)
    return jnp.where(x > 0, y, 0.0)


def _eas_body(ea_ref, out_ref):
    sel = (lax.broadcasted_iota(jnp.int32, (128, 32), 0) // 4
           == lax.broadcasted_iota(jnp.int32, (128, 32), 1)).astype(jnp.float32)
    out_ref[...] = jnp.dot(ea_ref[...], sel, preferred_element_type=jnp.float32)


def kernel(x, edge_index, edge_attr):
    n, d = x.shape
    e = edge_index.shape[1]
    dh = d // NC
    row = edge_index[0]
    col = edge_index[1]

    ept = ((e + NS * CHUNK - 1) // (NS * CHUNK)) * CHUNK
    nchunks = ept // CHUNK
    nchunks = ((nchunks + 15) // 16) * 16
    ept = nchunks * CHUNK
    e_pad = ept * NS
    nch_sc = nchunks // 2
    npt = ((n + NS * L - 1) // (NS * L)) * L
    n_pad = npt * NS

    row_p = jnp.pad(row, (0, e_pad - e)).reshape(NS, nchunks, CHUNK)
    col_p = jnp.pad(col, (0, e_pad - e)).reshape(NS, nchunks, CHUNK)
    ea_p = jnp.pad(edge_attr, ((0, e_pad - e), (0, 0)))
    rem = n % CHUNK
    nfull = n - rem

    eas = pl.pallas_call(
        _eas_body,
        out_shape=jax.ShapeDtypeStruct((e_pad // 32, 32), jnp.float32),
    )(ea_p.reshape(e_pad // 32, 128))
    eas_w = eas.reshape(NS, nchunks, CHUNK)

    def _prep_body(row_hbm, col_hbm, eas_hbm, w_hbm,
                   col_v, eas_v, row_v, dis_v, w_v, sml, acc_sh, dis_sh):
        cid = lax.axis_index("c")
        sid = lax.axis_index("s")

        def zf(i, _):
            sml[pl.ds(i * L, L)] = jnp.zeros((L,), jnp.float32)
            return ()
        lax.fori_loop(0, npt // L, zf, (), unroll=False)
        pltpu.sync_copy(sml, acc_sh.at[pl.ds(sid * npt, npt)])
        plsc.subcore_barrier()

        pltpu.sync_copy(col_hbm.at[sid], col_v)
        pltpu.sync_copy(eas_hbm.at[sid], eas_v)

        def dbody(j, _):
            pltpu.sync_copy(eas_v.at[j], acc_sh.at[col_v.at[j]], add=True)
            return ()
        lax.fori_loop(0, nchunks, dbody, (), unroll=False)
        plsc.subcore_barrier()

        pltpu.sync_copy(acc_sh.at[pl.ds(sid * npt, npt)], sml)

        def rbody(i, _):
            sl = pl.ds(i * L, L)
            sml[sl] = _rsqrt_pos(sml[sl])
            return ()
        lax.fori_loop(0, npt // L, rbody, (), unroll=False)
        pltpu.sync_copy(sml, dis_sh.at[pl.ds(sid * npt, npt)])
        plsc.subcore_barrier()
        pltpu.sync_copy(dis_sh, dis_v)

        cbase = cid * nch_sc
        pltpu.sync_copy(row_hbm.at[sid].at[pl.ds(cbase, nch_sc)], row_v)

        def wbody(j, _):
            for g in range(CHUNK // L):
                sl = pl.ds(g * L, L)
                r16 = row_v[j, sl]
                c16 = col_v[cbase + j, sl]
                dr = plsc.load_gather(dis_v, [r16])
                dc = plsc.load_gather(dis_v, [c16])
                w_v[j, sl] = dr * eas_v[cbase + j, sl] * dc
            return ()
        lax.fori_loop(0, nch_sc, wbody, (), unroll=False)
        pltpu.sync_copy(w_v, w_hbm.at[sid].at[pl.ds(cbase, nch_sc)])

    w = pl.kernel(
        _prep_body,
        out_type=jax.ShapeDtypeStruct((NS, nchunks, CHUNK), jnp.float32),
        mesh=plsc.VectorSubcoreMesh(**_MESH),
        compiler_params=pltpu.CompilerParams(**_CP),
        scratch_types=[
            pltpu.VMEM((nchunks, CHUNK), jnp.int32),
            pltpu.VMEM((nchunks, CHUNK), jnp.float32),
            pltpu.VMEM((nch_sc, CHUNK), jnp.int32),
            pltpu.VMEM((n_pad,), jnp.float32),
            pltpu.VMEM((nch_sc, CHUNK), jnp.float32),
            pltpu.VMEM((npt,), jnp.float32),
            pltpu.VMEM_SHARED((n_pad,), jnp.float32),
            pltpu.VMEM_SHARED((n_pad,), jnp.float32),
        ],
    )(row_p, col_p, eas_w)

    NBUF = 4
    NSEG = 2
    nch_seg = nchunks // NSEG

    def _hops_body(x_hbm, row_hbm, col_hbm, w_hbm, split_hbm, final_hbm,
                   row_v, col_v, w_v, gbufs, sbufs, gsems, ssems, acc_sh):
        cid = lax.axis_index("c")
        sid = lax.axis_index("s")
        cs = pl.ds(cid * dh, dh)
        ie = lax.broadcasted_iota(jnp.int32, (L,), 0) * 2
        io = ie + 1

        def to_bf16(sbuf, gbuf):
            @plsc.parallel_loop(0, CHUNK, step=1, unroll=8)
            def _(r):
                for dd in range(dh // (2 * L)):
                    a = plsc.load_gather(sbuf, [_full16(r), ie + dd * 2 * L])
                    b = plsc.load_gather(sbuf, [_full16(r), io + dd * 2 * L])
                    gbuf[r, pl.ds(dd * 2 * L, 2 * L)] = plsc.pack(
                        a, b, format=plsc.PackFormat.INTERLEAVED)

        for i in range(npt // CHUNK):
            start = sid * npt + i * CHUNK

            @pl.when(start + CHUNK <= n)
            def _():
                pltpu.sync_copy(x_hbm.at[pl.ds(start, CHUNK), cs], sbufs[0])
                to_bf16(sbufs[0], gbufs[0])
                pltpu.sync_copy(gbufs[0],
                                split_hbm.at[0, cid, pl.ds(start, CHUNK)])
                pltpu.sync_copy(sbufs[0],
                                final_hbm.at[0, pl.ds(start, CHUNK), cs])
            if rem:
                @pl.when(start == nfull)
                def _():
                    pltpu.sync_copy(x_hbm.at[pl.ds(start, rem), cs],
                                    sbufs[0].at[pl.ds(0, rem)])
                    to_bf16(sbufs[0], gbufs[0])
                    pltpu.sync_copy(gbufs[0].at[pl.ds(0, rem)],
                                    split_hbm.at[0, cid, pl.ds(start, rem)])
                    pltpu.sync_copy(sbufs[0].at[pl.ds(0, rem)],
                                    final_hbm.at[0, pl.ds(start, rem), cs])
        plsc.subcore_barrier()

        def scale(j, gbuf, sbuf):
            @plsc.parallel_loop(0, CHUNK, step=1, unroll=8)
            def _(r):
                wb = plsc.load_gather(w_v, [_full16(j), _full16(r)])
                for dd in range(dh // (2 * L)):
                    v = gbuf[r, pl.ds(dd * 2 * L, 2 * L)]
                    a, b = plsc.unpack(v, format=plsc.PackFormat.INTERLEAVED)
                    plsc.store_scatter(
                        sbuf, [_full16(r), ie + dd * 2 * L], a * wb)
                    plsc.store_scatter(
                        sbuf, [_full16(r), io + dd * 2 * L], b * wb)

        def hop(k, _):
            src = split_hbm.at[k - 1].at[cid]

            def zfill(i, _):
                for g in range(dh // L):
                    sbufs[0][i, pl.ds(g * L, L)] = jnp.zeros((L,),
                                                             jnp.float32)
                return ()
            lax.fori_loop(0, CHUNK, zfill, (), unroll=False)

            def zbody(i, _):
                pltpu.sync_copy(
                    sbufs[0], acc_sh.at[pl.ds(sid * npt + i * CHUNK, CHUNK)])
                return ()
            lax.fori_loop(0, npt // CHUNK, zbody, (), unroll=False)
            plsc.subcore_barrier()

            for seg in range(NSEG):
                cbase = seg * nch_seg
                pltpu.sync_copy(row_hbm.at[sid].at[pl.ds(cbase, nch_seg)],
                                row_v)
                pltpu.sync_copy(col_hbm.at[sid].at[pl.ds(cbase, nch_seg)],
                                col_v)
                pltpu.sync_copy(w_hbm.at[sid].at[pl.ds(cbase, nch_seg)], w_v)

                for b in range(NBUF - 1):
                    pltpu.async_copy(src.at[row_v.at[b]], gbufs[b], gsems[b])

                def quad(jj, _):
                    for b in range(NBUF):
                        j = jj * NBUF + b
                        pb = (b + NBUF - 1) % NBUF

                        if b == 0:
                            pltpu.async_copy(
                                src.at[row_v.at[j + NBUF - 1]], gbufs[pb],
                                gsems[pb])
                        else:
                            @pl.when(jj < nch_seg // NBUF - 1)
                            def _():
                                pltpu.async_copy(
                                    src.at[row_v.at[j + NBUF - 1]],
                                    gbufs[pb], gsems[pb])

                        pltpu.make_async_copy(
                            src.at[row_v.at[j]], gbufs[b], gsems[b]).wait()

                        @pl.when(jj >= 1)
                        def _():
                            pltpu.make_async_copy(
                                sbufs[b], acc_sh.at[col_v.at[j - NBUF]],
                                ssems[b]).wait()

                        scale(j, gbufs[b], sbufs[b])
                        pltpu.async_copy(sbufs[b], acc_sh.at[col_v.at[j]],
                                         ssems[b], add=True)
                    return ()
                lax.fori_loop(0, nch_seg // NBUF, quad, (), unroll=False)
                for b in range(NBUF):
                    pltpu.make_async_copy(
                        sbufs[b],
                        acc_sh.at[col_v.at[nch_seg - NBUF + b]],
                        ssems[b]).wait()

            plsc.subcore_barrier()

            for i in range(npt // CHUNK):
                start = sid * npt + i * CHUNK
                pltpu.sync_copy(acc_sh.at[pl.ds(start, CHUNK)], sbufs[0])

                @pl.when(k < K_HOPS)
                def _():
                    to_bf16(sbufs[0], gbufs[0])
                    pltpu.sync_copy(gbufs[0],
                                    split_hbm.at[k, cid, pl.ds(start, CHUNK)])

                @pl.when(start + CHUNK <= n)
                def _():
                    pltpu.sync_copy(sbufs[0],
                                    final_hbm.at[k, pl.ds(start, CHUNK), cs])
                if rem:
                    @pl.when(start == nfull)
                    def _():
                        pltpu.sync_copy(
                            sbufs[0].at[pl.ds(0, rem)],
                            final_hbm.at[k, pl.ds(start, rem), cs])
            plsc.subcore_barrier()
            return ()
        lax.fori_loop(1, K_HOPS + 1, hop, (), unroll=False)

    _split, final = pl.kernel(
        _hops_body,
        out_type=(
            jax.ShapeDtypeStruct((K_HOPS, NC, n_pad, dh), jnp.bfloat16),
            jax.ShapeDtypeStruct((K_HOPS + 1, n, d), jnp.float32),
        ),
        mesh=plsc.VectorSubcoreMesh(**_MESH),
        compiler_params=pltpu.CompilerParams(**_CP),
        scratch_types=[
            pltpu.VMEM((nch_seg, CHUNK), jnp.int32),
            pltpu.VMEM((nch_seg, CHUNK), jnp.int32),
            pltpu.VMEM((nch_seg, CHUNK), jnp.float32),
            [pltpu.VMEM((CHUNK, dh), jnp.bfloat16) for _ in range(NBUF)],
            [pltpu.VMEM((CHUNK, dh), jnp.float32) for _ in range(NBUF)],
            [pltpu.SemaphoreType.DMA for _ in range(NBUF)],
            [pltpu.SemaphoreType.DMA for _ in range(NBUF)],
            pltpu.VMEM_SHARED((n_pad, dh), jnp.float32),
        ],
    )(x, row_p, col_p, w)
    return final
